# Initial kernel scaffold; baseline (speedup 1.0000x reference)
import jax
import jax.numpy as jnp
from jax.experimental import pallas as pl


def kernel(c1, c2, c3, c4, bn1_g, bn1_b, bn2_g, bn2_b, bn3_g, bn3_b, bn4_g, bn4_b, fc1_w, fc1_b, fc2_w, fc2_b, fc3_w, fc3_b, mu_w, mu_b, logvar_w, logvar_b, x, eps):
    raise NotImplementedError("write your pallas kernel here")


# speedup vs baseline: 20.6384x; 20.6384x over previous
import jax
import jax.numpy as jnp
from jax.experimental import pallas as pl
from jax.experimental.pallas import tpu as pltpu

SLOPE = 0.2
BN_EPS = 1e-5
F32 = jnp.float32
BF16 = jnp.bfloat16


def _leaky(v):
    return jnp.where(v >= 0, v, SLOPE * v)


def _conv1_body(p_ref, w_ref, o_ref, ps_ref):
    oh, ow, bn, kk = p_ref.shape
    cout = o_ref.shape[3]
    x = p_ref[...].reshape(oh * ow * bn, kk)
    y = jnp.dot(x, w_ref[...], preferred_element_type=F32)
    o_ref[...] = y.reshape(oh, ow, bn, cout)
    s = jnp.zeros((1, cout), F32)
    ss = jnp.zeros((1, cout), F32)
    for oy in range(oh):
        v = o_ref[oy].reshape(ow * bn, cout)
        s = s + jnp.sum(v, axis=0, keepdims=True)
        ss = ss + jnp.sum(v * v, axis=0, keepdims=True)
    ps_ref[0] = jnp.concatenate([s, ss], axis=0)


def _conv_body(h_ref, a_ref, b_ref, w_ref, o_ref, ps_ref, *s_refs):
    hin, win, bn, cin = h_ref.shape
    oh, ow, _, cout = o_ref.shape
    nck = len(s_refs)
    a = a_ref[0]
    b = b_ref[0]
    for iy in range(hin):
        v = _leaky(h_ref[iy] * a + b)
        for c in range(nck):
            s_refs[c][iy] = v[:, :, c * 128:(c + 1) * 128]

    taps = [(1, 1)] + [(di, dj) for di in range(4) for dj in range(4)
                       if (di, dj) != (1, 1)]
    for di, dj in taps:
        iy0, ix0 = di - 1, dj - 1
        ylo = max(0, (-iy0 + 1) // 2)
        yhi = min(oh, (hin - 1 - iy0) // 2 + 1)
        xlo = max(0, (-ix0 + 1) // 2)
        xhi = min(ow, (win - 1 - ix0) // 2 + 1)
        ny, nx = yhi - ylo, xhi - xlo
        m = None
        for c in range(nck):
            src = s_refs[c][pl.ds(2 * ylo + iy0, ny, 2),
                            pl.ds(2 * xlo + ix0, nx, 2)]
            src = src.astype(BF16).reshape(ny * nx * bn, 128)
            mc = jnp.dot(src, w_ref[di * 4 + dj, c * 128:(c + 1) * 128],
                         preferred_element_type=F32)
            m = mc if m is None else m + mc
        m = m.reshape(ny, nx, bn, cout)
        if (di, dj) == (1, 1):
            o_ref[...] = m
        else:
            o_ref[ylo:yhi, xlo:xhi] += m

    s = jnp.zeros((1, cout), F32)
    ss = jnp.zeros((1, cout), F32)
    for oy in range(oh):
        v = o_ref[oy].reshape(ow * bn, cout)
        s = s + jnp.sum(v, axis=0, keepdims=True)
        ss = ss + jnp.sum(v * v, axis=0, keepdims=True)
    ps_ref[0] = jnp.concatenate([s, ss], axis=0)


def _mlp_body(h_ref, a_ref, b_ref, w1_ref, b1_ref, w2_ref, b2_ref,
              w3_ref, b3_ref, wm_ref, bm_ref, wl_ref, bl_ref, eps_ref,
              z_ref, mu_ref, lv_ref, acc_ref):
    a = a_ref[0]
    b = b_ref[0]
    for s in range(9):
        y, x = divmod(s, 3)
        v = _leaky(h_ref[y, x] * a + b).astype(BF16)
        m = jnp.dot(v, w1_ref[s], preferred_element_type=F32)
        if s == 0:
            acc_ref[...] = m
        else:
            acc_ref[...] += m
    h1 = _leaky(acc_ref[...] + b1_ref[...]).astype(BF16)
    h2 = _leaky(jnp.dot(h1, w2_ref[...], preferred_element_type=F32)
                + b2_ref[...]).astype(BF16)
    h3 = _leaky(jnp.dot(h2, w3_ref[...], preferred_element_type=F32)
                + b3_ref[...]).astype(BF16)
    mu = jnp.dot(h3, wm_ref[...], preferred_element_type=F32) + bm_ref[...]
    lv = jnp.dot(h3, wl_ref[...], preferred_element_type=F32) + bl_ref[...]
    mu_ref[...] = mu
    lv_ref[...] = lv
    z_ref[...] = mu + jnp.exp(0.5 * lv) * eps_ref[...]


def _ab(ps, gamma, beta, count):
    s = jnp.sum(ps[:, 0, :], axis=0)
    ss = jnp.sum(ps[:, 1, :], axis=0)
    mean = s / count
    var = ss / count - mean * mean
    a = gamma * jax.lax.rsqrt(var + BN_EPS)
    b = beta - mean * a
    return a.reshape(1, -1), b.reshape(1, -1)


def _conv_layer(h, aa, bb, w, bn):
    hin, win, n, cin = h.shape
    cout = w.shape[0]
    oh = (hin - 2) // 2 + 1
    g = n // bn
    wt = jnp.transpose(w, (2, 3, 1, 0)).reshape(16, cin, cout).astype(BF16)
    return pl.pallas_call(
        _conv_body,
        out_shape=(jax.ShapeDtypeStruct((oh, oh, n, cout), F32),
                   jax.ShapeDtypeStruct((g, 2, cout), F32)),
        grid=(g,),
        in_specs=[
            pl.BlockSpec((hin, win, bn, cin), lambda i: (0, 0, i, 0)),
            pl.BlockSpec((1, cin), lambda i: (0, 0)),
            pl.BlockSpec((1, cin), lambda i: (0, 0)),
            pl.BlockSpec((16, cin, cout), lambda i: (0, 0, 0)),
        ],
        out_specs=(pl.BlockSpec((oh, oh, bn, cout), lambda i: (0, 0, i, 0)),
                   pl.BlockSpec((1, 2, cout), lambda i: (i, 0, 0))),
        scratch_shapes=[pltpu.VMEM((hin, win, bn, 128), F32)
                        for _ in range(cin // 128)],
        compiler_params=pltpu.CompilerParams(
            dimension_semantics=("parallel",)),
    )(h, aa, bb, wt)


def kernel(c1, c2, c3, c4, bn1_g, bn1_b, bn2_g, bn2_b, bn3_g, bn3_b,
           bn4_g, bn4_b, fc1_w, fc1_b, fc2_w, fc2_b, fc3_w, fc3_b,
           mu_w, mu_b, logvar_w, logvar_b, x, eps):
    n = x.shape[0]
    bn = 32 if n % 32 == 0 else (16 if n % 16 == 0 else 8)
    d = c1.shape[0]
    latent = mu_w.shape[1]

    xt = jnp.transpose(x[:, 0], (1, 2, 0))
    p = jnp.stack([xt[di:di + 25, dj:dj + 25]
                   for di in range(4) for dj in range(4)],
                  axis=-1).astype(BF16)
    w1 = jnp.transpose(c1[:, 0], (1, 2, 0)).reshape(16, d).astype(BF16)
    g = n // bn
    h1, ps1 = pl.pallas_call(
        _conv1_body,
        out_shape=(jax.ShapeDtypeStruct((25, 25, n, d), F32),
                   jax.ShapeDtypeStruct((g, 2, d), F32)),
        grid=(g,),
        in_specs=[pl.BlockSpec((25, 25, bn, 16), lambda i: (0, 0, i, 0)),
                  pl.BlockSpec((16, d), lambda i: (0, 0))],
        out_specs=(pl.BlockSpec((25, 25, bn, d), lambda i: (0, 0, i, 0)),
                   pl.BlockSpec((1, 2, d), lambda i: (i, 0, 0))),
        compiler_params=pltpu.CompilerParams(
            dimension_semantics=("parallel",)),
    )(p, w1)

    a1, s1 = _ab(ps1, bn1_g, bn1_b, n * 625)
    h2, ps2 = _conv_layer(h1, a1, s1, c2, bn)
    a2, s2 = _ab(ps2, bn2_g, bn2_b, n * 144)
    h3, ps3 = _conv_layer(h2, a2, s2, c3, bn)
    a3, s3 = _ab(ps3, bn3_g, bn3_b, n * 36)
    h4, ps4 = _conv_layer(h3, a3, s3, c4, bn)
    a4, s4 = _ab(ps4, bn4_g, bn4_b, n * 9)

    c8 = 8 * d
    w1r = fc1_w.reshape(c8, 9, c8).transpose(1, 0, 2).astype(BF16)
    bm = 128 if n % 128 == 0 else n
    gm = n // bm
    z, mu, lv = pl.pallas_call(
        _mlp_body,
        out_shape=(jax.ShapeDtypeStruct((n, latent), F32),
                   jax.ShapeDtypeStruct((n, latent), F32),
                   jax.ShapeDtypeStruct((n, latent), F32)),
        grid=(gm,),
        in_specs=[
            pl.BlockSpec((3, 3, bm, c8), lambda i: (0, 0, i, 0)),
            pl.BlockSpec((1, c8), lambda i: (0, 0)),
            pl.BlockSpec((1, c8), lambda i: (0, 0)),
            pl.BlockSpec((9, c8, c8), lambda i: (0, 0, 0)),
            pl.BlockSpec((1, c8), lambda i: (0, 0)),
            pl.BlockSpec((c8, 4 * d), lambda i: (0, 0)),
            pl.BlockSpec((1, 4 * d), lambda i: (0, 0)),
            pl.BlockSpec((4 * d, d), lambda i: (0, 0)),
            pl.BlockSpec((1, d), lambda i: (0, 0)),
            pl.BlockSpec((d, latent), lambda i: (0, 0)),
            pl.BlockSpec((1, latent), lambda i: (0, 0)),
            pl.BlockSpec((d, latent), lambda i: (0, 0)),
            pl.BlockSpec((1, latent), lambda i: (0, 0)),
            pl.BlockSpec((bm, latent), lambda i: (i, 0)),
        ],
        out_specs=(pl.BlockSpec((bm, latent), lambda i: (i, 0)),
                   pl.BlockSpec((bm, latent), lambda i: (i, 0)),
                   pl.BlockSpec((bm, latent), lambda i: (i, 0))),
        scratch_shapes=[pltpu.VMEM((bm, c8), F32)],
        compiler_params=pltpu.CompilerParams(
            dimension_semantics=("parallel",)),
    )(h4, a4, s4, w1r, fc1_b.reshape(1, -1),
      fc2_w.astype(BF16), fc2_b.reshape(1, -1),
      fc3_w.astype(BF16), fc3_b.reshape(1, -1),
      mu_w.astype(BF16), mu_b.reshape(1, -1),
      logvar_w.astype(BF16), logvar_b.reshape(1, -1), eps)
    return z, mu, lv
